# baseline (speedup 1.0000x reference)
import functools

import numpy as np

import jax
import jax.numpy as jnp
from jax.experimental import pallas as pl
from jax.experimental.pallas import tpu as pltpu

_VMEM_LIMIT = 64 * 1024 * 1024


def _taps_3x3(a, H, W, m0, mW):
    HW, C = a.shape
    z1 = jnp.zeros((1, C), a.dtype)
    bm1 = jnp.concatenate([z1, a[:-1]], axis=0) * m0
    bp1 = jnp.concatenate([a[1:], z1], axis=0) * mW
    bases = (bm1, a, bp1)
    cols = []
    for oy in (-1, 0, 1):
        for b in bases:
            s = oy * W
            if s > 0:
                t = jnp.concatenate([b[s:], jnp.zeros((s, C), a.dtype)], axis=0)
            elif s < 0:
                t = jnp.concatenate([jnp.zeros((-s, C), a.dtype), b[:s]], axis=0)
            else:
                t = b
            cols.append(t)
    return jnp.concatenate(cols, axis=1)


def _stage_kernel(x_ref, sk_ref, w0_ref, w1_ref, wu_ref, wp_ref,
                  b0_ref, b1_ref, bt_ref, bp_ref, m_ref, o_ref, y_scr,
                  *, H, W, KCH):
    HW = H * W
    C = x_ref.shape[-1]
    k = pl.program_id(1)

    @pl.when(k == 0)
    def _convs():
        x = x_ref[0].reshape(HW, C)
        m0 = m_ref[:, :C]
        mW = m_ref[:, C:]
        c0 = _taps_3x3(x.astype(jnp.bfloat16), H, W, m0, mW)
        h = jnp.dot(c0, w0_ref[...].astype(jnp.bfloat16),
                    preferred_element_type=jnp.float32)
        h = jnp.maximum(h + b0_ref[...], 0.0)
        c1 = _taps_3x3(h.astype(jnp.bfloat16), H, W, m0, mW)
        y = jnp.dot(c1, w1_ref[...].astype(jnp.bfloat16),
                    preferred_element_type=jnp.float32)
        y_scr[...] = y + b1_ref[...] + x

    Hc = H // KCH
    rows = Hc * W
    yk = y_scr[pl.ds(k * rows, rows), :]
    ua = jnp.dot(yk.astype(jnp.bfloat16), wu_ref[...].astype(jnp.bfloat16),
                 preferred_element_type=jnp.float32)
    ua = ua + bt_ref[...]
    Cout = ua.shape[1] // 4
    uq = [ua[:, q * Cout:(q + 1) * Cout] for q in range(4)]
    for ky in range(2):
        r = jnp.stack([uq[2 * ky], uq[2 * ky + 1]],
                      axis=1).reshape(2 * rows, Cout)
        sk = sk_ref[0, :, ky, :, :].reshape(2 * rows, -1)
        pr = jnp.dot(sk.astype(jnp.bfloat16), wp_ref[...].astype(jnp.bfloat16),
                     preferred_element_type=jnp.float32)
        pr = jnp.maximum(pr + bp_ref[...], 0.0)
        o_ref[0, :, ky, :, :] = (r + pr).reshape(
            Hc, 2 * W, Cout).astype(o_ref.dtype)


def kernel(inp, skip, w0, b0, w1, b1, wt, bt, wp, bp):
    N, C, H, W = inp.shape
    Csk = skip.shape[1]
    Cout = wt.shape[-1]
    HW = H * W
    x_nhwc = jnp.transpose(inp, (0, 2, 3, 1))
    sk5 = jnp.transpose(skip, (0, 2, 3, 1)).reshape(N, H, 2, 2 * W, Csk)
    w0f = w0.reshape(9 * C, C)
    w1f = w1.reshape(9 * C, C)
    wu = wt.reshape(C, 4 * Cout)
    wpb = wp
    b0r = b0.reshape(1, C)
    b1r = b1.reshape(1, C)
    btr = jnp.tile(bt.reshape(1, Cout), (1, 4))
    bpr = bp.reshape(1, Cout)
    col = np.arange(HW, dtype=np.int32) % W
    m01 = np.broadcast_to(
        np.stack([col != 0, col != W - 1], 0)[:, :, None],
        (2, HW, C)).astype(jnp.bfloat16)
    m01 = jnp.asarray(np.concatenate([m01[0], m01[1]], axis=1))
    KCH = 2
    Hc = H // KCH
    out5 = pl.pallas_call(
        functools.partial(_stage_kernel, H=H, W=W, KCH=KCH),
        out_shape=jax.ShapeDtypeStruct((N, H, 2, 2 * W, Cout), inp.dtype),
        grid=(N, KCH),
        in_specs=[
            pl.BlockSpec((1, H, W, C), lambda n, k: (n, 0, 0, 0)),
            pl.BlockSpec((1, Hc, 2, 2 * W, Csk), lambda n, k: (n, k, 0, 0, 0)),
            pl.BlockSpec((9 * C, C), lambda n, k: (0, 0)),
            pl.BlockSpec((9 * C, C), lambda n, k: (0, 0)),
            pl.BlockSpec((C, 4 * Cout), lambda n, k: (0, 0)),
            pl.BlockSpec((Csk, Cout), lambda n, k: (0, 0)),
            pl.BlockSpec((1, C), lambda n, k: (0, 0)),
            pl.BlockSpec((1, C), lambda n, k: (0, 0)),
            pl.BlockSpec((1, 4 * Cout), lambda n, k: (0, 0)),
            pl.BlockSpec((1, Cout), lambda n, k: (0, 0)),
            pl.BlockSpec((HW, 2 * C), lambda n, k: (0, 0)),
        ],
        out_specs=pl.BlockSpec((1, Hc, 2, 2 * W, Cout),
                               lambda n, k: (n, k, 0, 0, 0)),
        scratch_shapes=[pltpu.VMEM((HW, C), jnp.float32)],
        compiler_params=pltpu.CompilerParams(
            dimension_semantics=("arbitrary", "arbitrary"),
            vmem_limit_bytes=_VMEM_LIMIT),
    )(x_nhwc, sk5, w0f, w1f, wu, wpb, b0r, b1r, btr, bpr, m01)
    out = out5.reshape(N, 2 * H, 2 * W, Cout)
    return jnp.transpose(out, (0, 3, 1, 2))


# speedup vs baseline: 1.1478x; 1.1478x over previous
import functools

import numpy as np

import jax
import jax.numpy as jnp
from jax.experimental import pallas as pl
from jax.experimental.pallas import tpu as pltpu

_VMEM_LIMIT = 64 * 1024 * 1024


def _taps_3x3(a, H, W, m0, mW):
    HW, C = a.shape
    z1 = jnp.zeros((1, C), a.dtype)
    bm1 = jnp.concatenate([z1, a[:-1]], axis=0) * m0
    bp1 = jnp.concatenate([a[1:], z1], axis=0) * mW
    bases = (bm1, a, bp1)
    cols = []
    for oy in (-1, 0, 1):
        for b in bases:
            s = oy * W
            if s > 0:
                t = jnp.concatenate([b[s:], jnp.zeros((s, C), a.dtype)], axis=0)
            elif s < 0:
                t = jnp.concatenate([jnp.zeros((-s, C), a.dtype), b[:s]], axis=0)
            else:
                t = b
            cols.append(t)
    return jnp.concatenate(cols, axis=1)


def _stage_kernel(x_ref, sk_ref, w0_ref, w1_ref, wu_ref, wp_ref,
                  b0_ref, b1_ref, bt_ref, bp_ref, m_ref, o_ref, *, H, W):
    HW = H * W
    x = x_ref[0].reshape(HW, -1)
    C = x.shape[1]
    m0 = m_ref[:, :C]
    mW = m_ref[:, C:]
    c0 = _taps_3x3(x.astype(jnp.bfloat16), H, W, m0, mW)
    h = jnp.dot(c0, w0_ref[...].astype(jnp.bfloat16),
                preferred_element_type=jnp.float32)
    h = jnp.maximum(h + b0_ref[...], 0.0)
    c1 = _taps_3x3(h.astype(jnp.bfloat16), H, W, m0, mW)
    y = jnp.dot(c1, w1_ref[...].astype(jnp.bfloat16),
                preferred_element_type=jnp.float32)
    y = y + b1_ref[...] + x
    ua = jnp.dot(y.astype(jnp.bfloat16), wu_ref[...].astype(jnp.bfloat16),
                 preferred_element_type=jnp.float32)
    ua = ua + bt_ref[...]
    Cout = ua.shape[1] // 4
    uq = [ua[:, q * Cout:(q + 1) * Cout] for q in range(4)]
    sk = sk_ref[0].reshape(4 * HW, -1)
    pr = jnp.dot(sk.astype(jnp.bfloat16), wp_ref[...].astype(jnp.bfloat16),
                 preferred_element_type=jnp.float32)
    pr = jnp.maximum(pr + bp_ref[...], 0.0)
    pr4 = pr.reshape(H, 2, 2 * W, Cout)
    for ky in range(2):
        r = jnp.stack([uq[2 * ky], uq[2 * ky + 1]],
                      axis=1).reshape(2 * HW, Cout)
        prk = pr4[:, ky, :, :].reshape(2 * HW, Cout)
        o_ref[0, :, ky, :, :] = (r + prk).reshape(
            H, 2 * W, Cout).astype(o_ref.dtype)


def kernel(inp, skip, w0, b0, w1, b1, wt, bt, wp, bp):
    N, C, H, W = inp.shape
    Csk = skip.shape[1]
    Cout = wt.shape[-1]
    HW = H * W
    x_nhwc = jnp.transpose(inp, (0, 2, 3, 1))
    sk5 = jnp.transpose(skip, (0, 2, 3, 1)).reshape(N, H, 2, 2 * W, Csk)
    w0f = w0.reshape(9 * C, C)
    w1f = w1.reshape(9 * C, C)
    wu = wt.reshape(C, 4 * Cout)
    wpb = wp
    b0r = b0.reshape(1, C)
    b1r = b1.reshape(1, C)
    btr = jnp.tile(bt.reshape(1, Cout), (1, 4))
    bpr = bp.reshape(1, Cout)
    col = np.arange(HW, dtype=np.int32) % W
    m01 = np.broadcast_to(
        np.stack([col != 0, col != W - 1], 0)[:, :, None],
        (2, HW, C)).astype(jnp.bfloat16)
    m01 = jnp.asarray(np.concatenate([m01[0], m01[1]], axis=1))
    out5 = pl.pallas_call(
        functools.partial(_stage_kernel, H=H, W=W),
        out_shape=jax.ShapeDtypeStruct((N, H, 2, 2 * W, Cout), inp.dtype),
        grid=(N,),
        in_specs=[
            pl.BlockSpec((1, H, W, C), lambda n: (n, 0, 0, 0)),
            pl.BlockSpec((1, H, 2, 2 * W, Csk), lambda n: (n, 0, 0, 0, 0)),
            pl.BlockSpec((9 * C, C), lambda n: (0, 0)),
            pl.BlockSpec((9 * C, C), lambda n: (0, 0)),
            pl.BlockSpec((C, 4 * Cout), lambda n: (0, 0)),
            pl.BlockSpec((Csk, Cout), lambda n: (0, 0)),
            pl.BlockSpec((1, C), lambda n: (0, 0)),
            pl.BlockSpec((1, C), lambda n: (0, 0)),
            pl.BlockSpec((1, 4 * Cout), lambda n: (0, 0)),
            pl.BlockSpec((1, Cout), lambda n: (0, 0)),
            pl.BlockSpec((HW, 2 * C), lambda n: (0, 0)),
        ],
        out_specs=pl.BlockSpec((1, H, 2, 2 * W, Cout),
                               lambda n: (n, 0, 0, 0, 0)),
        compiler_params=pltpu.CompilerParams(
            dimension_semantics=("arbitrary",),
            vmem_limit_bytes=_VMEM_LIMIT),
    )(x_nhwc, sk5, w0f, w1f, wu, wpb, b0r, b1r, btr, bpr, m01)
    out = out5.reshape(N, 2 * H, 2 * W, Cout)
    return jnp.transpose(out, (0, 3, 1, 2))
